# baseline (speedup 1.0000x reference)
import functools

import jax
import jax.numpy as jnp
from jax import lax
from jax.experimental import pallas as pl
from jax.experimental.pallas import tpu as pltpu
from jax.experimental.pallas import tpu_sc as plsc

N_NODES = 10000
D = 128
N_PAD = 10240
NW = 32
NS = 16
CHUNK = 128
ROWS_PER_TILE = N_PAD // NS

_mesh = plsc.VectorSubcoreMesh(core_axis_name="c", subcore_axis_name="s")


def _hist_body(nchunks, dst_hbm, out_hbm, idx_v, ones_v, zbuf_v, hist_sh):
    s = lax.axis_index("s")
    c = lax.axis_index("c")
    w = c * NS + s

    def fill_ones(i, _):
        ones_v[pl.ds(i * 16, 16)] = jnp.ones((16,), jnp.float32)
        return 0
    lax.fori_loop(0, CHUNK // 16, fill_ones, 0)

    def fill_z(i, _):
        zbuf_v[pl.ds(i * 16, 16)] = jnp.zeros((16,), jnp.float32)
        return 0
    lax.fori_loop(0, ROWS_PER_TILE // 16, fill_z, 0)

    pltpu.sync_copy(zbuf_v, hist_sh.at[pl.ds(s * ROWS_PER_TILE, ROWS_PER_TILE)])
    plsc.subcore_barrier()

    pltpu.sync_copy(dst_hbm.at[w], idx_v)

    def chunk_step(j, _):
        pltpu.sync_copy(ones_v, hist_sh.at[idx_v.at[j]], add=True)
        return 0
    lax.fori_loop(0, nchunks, chunk_step, 0)
    plsc.subcore_barrier()

    pltpu.sync_copy(hist_sh.at[pl.ds(s * ROWS_PER_TILE, ROWS_PER_TILE)],
                    out_hbm.at[c, pl.ds(s * ROWS_PER_TILE, ROWS_PER_TILE)])


def _make_hist(nchunks):
    return functools.partial(
        pl.kernel,
        out_type=jax.ShapeDtypeStruct((2, N_PAD), jnp.float32),
        mesh=_mesh,
        scratch_types=[
            pltpu.VMEM((nchunks, CHUNK), jnp.int32),
            pltpu.VMEM((CHUNK,), jnp.float32),
            pltpu.VMEM((ROWS_PER_TILE,), jnp.float32),
            pltpu.VMEM_SHARED((N_PAD,), jnp.float32),
        ],
    )(functools.partial(_hist_body, nchunks))


def _scatter_body(n0, n1, src_hbm, dst_hbm, g_hbm, out_hbm,
                  sidx_v, didx_v, rows_v, acc_sh, gsem, isem, ssem):
    s = lax.axis_index("s")
    c = lax.axis_index("c")
    nchunks = lax.select(c == 0, n0, n1)
    base = lax.select(c == 0, s * n0, NS * n0 + s * n1)

    def fill_z(i, _):
        rows_v[0, i, :] = jnp.zeros((D,), jnp.float32)
        return 0
    lax.fori_loop(0, CHUNK, fill_z, 0)

    def zero_step(t, _):
        pltpu.sync_copy(rows_v.at[0],
                        acc_sh.at[pl.ds(s * ROWS_PER_TILE + t * CHUNK, CHUNK)])
        return 0
    lax.fori_loop(0, ROWS_PER_TILE // CHUNK, zero_step, 0)
    pltpu.sync_copy(rows_v.at[0, pl.ds(0, ROWS_PER_TILE % CHUNK)],
                    acc_sh.at[pl.ds(s * ROWS_PER_TILE
                                    + (ROWS_PER_TILE // CHUNK) * CHUNK,
                                    ROWS_PER_TILE % CHUNK)])
    plsc.subcore_barrier()

    pltpu.sync_copy(src_hbm.at[base], sidx_v.at[0])
    pltpu.sync_copy(dst_hbm.at[base], didx_v.at[0])

    @pl.when(nchunks > 1)
    def _():
        pltpu.async_copy(src_hbm.at[base + 1], sidx_v.at[1], isem)
        pltpu.async_copy(dst_hbm.at[base + 1], didx_v.at[1], isem)

    pltpu.async_copy(g_hbm.at[sidx_v.at[0]], rows_v.at[0], gsem)

    def chunk_step(j, _):
        cur = lax.rem(j, 2)
        nxt = lax.rem(j + 1, 2)
        dcur = lax.rem(j, 3)
        dnxt = lax.rem(j + 1, 3)

        @pl.when(j >= 1)
        def _():
            pltpu.make_async_copy(rows_v.at[nxt],
                                  acc_sh.at[didx_v.at[lax.rem(j + 2, 3)]],
                                  ssem).wait()

        @pl.when(j + 1 < nchunks)
        def _():
            pltpu.make_async_copy(src_hbm.at[base + j + 1], sidx_v.at[nxt], isem).wait()
            pltpu.make_async_copy(dst_hbm.at[base + j + 1], didx_v.at[dnxt], isem).wait()
            pltpu.async_copy(g_hbm.at[sidx_v.at[nxt]], rows_v.at[nxt], gsem)

        pltpu.make_async_copy(g_hbm.at[sidx_v.at[cur]], rows_v.at[cur], gsem).wait()
        pltpu.async_copy(rows_v.at[cur], acc_sh.at[didx_v.at[dcur]], ssem, add=True)

        @pl.when(j + 2 < nchunks)
        def _():
            pltpu.async_copy(src_hbm.at[base + j + 2], sidx_v.at[cur], isem)
            pltpu.async_copy(dst_hbm.at[base + j + 2], didx_v.at[lax.rem(j + 2, 3)], isem)
        return 0

    lax.fori_loop(0, nchunks, chunk_step, 0)
    pltpu.make_async_copy(rows_v.at[lax.rem(nchunks - 1, 2)],
                          acc_sh.at[didx_v.at[lax.rem(nchunks - 1, 3)]], ssem).wait()
    plsc.subcore_barrier()

    def dump_step(t, _):
        o = s * ROWS_PER_TILE + t * CHUNK
        pltpu.sync_copy(acc_sh.at[pl.ds(o, CHUNK)], out_hbm.at[c, pl.ds(o, CHUNK)])
        return 0
    lax.fori_loop(0, ROWS_PER_TILE // CHUNK, dump_step, 0)
    o = s * ROWS_PER_TILE + (ROWS_PER_TILE // CHUNK) * CHUNK
    r = ROWS_PER_TILE % CHUNK
    pltpu.sync_copy(acc_sh.at[pl.ds(o, r)], out_hbm.at[c, pl.ds(o, r)])


def _make_scatter(n0, n1):
    return functools.partial(
        pl.kernel,
        out_type=jax.ShapeDtypeStruct((2, N_PAD, D), jnp.float32),
        mesh=_mesh,
        scratch_types=[
            pltpu.VMEM((2, CHUNK), jnp.int32),
            pltpu.VMEM((3, CHUNK), jnp.int32),
            pltpu.VMEM((2, CHUNK, D), jnp.float32),
            pltpu.VMEM_SHARED((N_PAD, D), jnp.float32),
            pltpu.SemaphoreType.DMA,
            pltpu.SemaphoreType.DMA,
            pltpu.SemaphoreType.DMA,
        ],
    )(functools.partial(_scatter_body, n0, n1))


def _pre_body(x_ref, w_ref, histt_ref, g_ref):
    h = lax.dot_general(x_ref[...], w_ref[...],
                        (((1,), (1,)), ((), ())),
                        preferred_element_type=jnp.float32)
    deg = histt_ref[:, 0:1] + histt_ref[:, 1:2] + 1.0
    g_ref[...] = h * lax.rsqrt(deg)


def _post_body(pa_ref, pb_ref, g_ref, histt_ref, b_ref, out_ref):
    acc = pa_ref[...] + pb_ref[...] + g_ref[...]
    deg = histt_ref[:, 0:1] + histt_ref[:, 1:2] + 1.0
    out = acc * lax.rsqrt(deg) + b_ref[...]
    out_ref[...] = jnp.maximum(out, 0.0)


_BLK = 256
_GRID = N_PAD // _BLK


def _tc_pre(x_p, W, histt):
    return pl.pallas_call(
        _pre_body,
        grid=(_GRID,),
        in_specs=[
            pl.BlockSpec((_BLK, D), lambda i: (i, 0)),
            pl.BlockSpec((D, D), lambda i: (0, 0)),
            pl.BlockSpec((_BLK, 2), lambda i: (i, 0)),
        ],
        out_specs=pl.BlockSpec((_BLK, D), lambda i: (i, 0)),
        out_shape=jax.ShapeDtypeStruct((N_PAD, D), jnp.float32),
    )(x_p, W, histt)


def _tc_post(pa, pb, g, histt, b2d):
    return pl.pallas_call(
        _post_body,
        grid=(_GRID,),
        in_specs=[
            pl.BlockSpec((_BLK, D), lambda i: (i, 0)),
            pl.BlockSpec((_BLK, D), lambda i: (i, 0)),
            pl.BlockSpec((_BLK, D), lambda i: (i, 0)),
            pl.BlockSpec((_BLK, 2), lambda i: (i, 0)),
            pl.BlockSpec((1, D), lambda i: (0, 0)),
        ],
        out_specs=pl.BlockSpec((_BLK, D), lambda i: (i, 0)),
        out_shape=jax.ShapeDtypeStruct((N_PAD, D), jnp.float32),
    )(pa, pb, g, histt, b2d)


_SLOW_SHARE = 120 / 157


def kernel(x, edge_index, W, b):
    src = edge_index[0].astype(jnp.int32)
    dst = edge_index[1].astype(jnp.int32)
    E = src.shape[0]

    per_w = -(-E // NW)
    nch_h = -(-per_w // CHUNK)
    pad_h = NW * nch_h * CHUNK - E
    pad_dst_h = N_NODES + jnp.arange(pad_h, dtype=jnp.int32) % (N_PAD - N_NODES)
    dst_h = jnp.concatenate([dst, pad_dst_h]).reshape(NW, nch_h, CHUNK)

    per_tile = -(-(-(-E // CHUNK)) // NS)
    n0 = max(1, round(per_tile * _SLOW_SHARE))
    n1 = per_tile - n0
    e_pad = NS * per_tile * CHUNK
    pad_n = e_pad - E
    pad_dst = N_NODES + jnp.arange(pad_n, dtype=jnp.int32) % (N_PAD - N_NODES)
    src_p = jnp.pad(src, (0, pad_n)).reshape(NS * per_tile, CHUNK)
    dst_p = jnp.concatenate([dst, pad_dst]).reshape(NS * per_tile, CHUNK)

    x_p = jnp.pad(x, ((0, N_PAD - x.shape[0]), (0, 0)))

    hist = _make_hist(nch_h)(dst_h)
    histt = hist.T
    g = _tc_pre(x_p, W, histt)
    parts = _make_scatter(n0, n1)(src_p, dst_p, g)
    out = _tc_post(parts[0], parts[1], g, histt, b.reshape(1, D))
    return out[:x.shape[0]]


# speedup vs baseline: 1.0537x; 1.0537x over previous
import functools

import jax
import jax.numpy as jnp
from jax import lax
from jax.experimental import pallas as pl
from jax.experimental.pallas import tpu as pltpu
from jax.experimental.pallas import tpu_sc as plsc

N_NODES = 10000
D = 128
N_PAD = 10240
NW = 32
NS = 16
CHUNK = 128
ROWS_PER_TILE = N_PAD // NS

_mesh = plsc.VectorSubcoreMesh(core_axis_name="c", subcore_axis_name="s")


def _hist_body(nchunks, dst_hbm, out_hbm, idx_v, ones_v, zbuf_v, hist_sh):
    s = lax.axis_index("s")
    c = lax.axis_index("c")
    w = c * NS + s

    def fill_ones(i, _):
        ones_v[pl.ds(i * 16, 16)] = jnp.ones((16,), jnp.float32)
        return 0
    lax.fori_loop(0, CHUNK // 16, fill_ones, 0)

    def fill_z(i, _):
        zbuf_v[pl.ds(i * 16, 16)] = jnp.zeros((16,), jnp.float32)
        return 0
    lax.fori_loop(0, ROWS_PER_TILE // 16, fill_z, 0)

    pltpu.sync_copy(zbuf_v, hist_sh.at[pl.ds(s * ROWS_PER_TILE, ROWS_PER_TILE)])
    plsc.subcore_barrier()

    pltpu.sync_copy(dst_hbm.at[w], idx_v)

    def chunk_step(j, _):
        pltpu.sync_copy(ones_v, hist_sh.at[idx_v.at[j]], add=True)
        return 0
    lax.fori_loop(0, nchunks, chunk_step, 0)
    plsc.subcore_barrier()

    pltpu.sync_copy(hist_sh.at[pl.ds(s * ROWS_PER_TILE, ROWS_PER_TILE)],
                    out_hbm.at[c, pl.ds(s * ROWS_PER_TILE, ROWS_PER_TILE)])


def _make_hist(nchunks):
    return functools.partial(
        pl.kernel,
        out_type=jax.ShapeDtypeStruct((2, N_PAD), jnp.float32),
        mesh=_mesh,
        scratch_types=[
            pltpu.VMEM((nchunks, CHUNK), jnp.int32),
            pltpu.VMEM((CHUNK,), jnp.float32),
            pltpu.VMEM((ROWS_PER_TILE,), jnp.float32),
            pltpu.VMEM_SHARED((N_PAD,), jnp.float32),
        ],
    )(functools.partial(_hist_body, nchunks))


def _scatter_body(n0, n1, src_hbm, dst_hbm, g_hbm, out_hbm,
                  sidx_v, didx_v, rows_v, acc_sh, gsem, isem, ssem):
    s = lax.axis_index("s")
    c = lax.axis_index("c")
    nchunks = lax.select(c == 0, n0, n1)
    base = lax.select(c == 0, s * n0, NS * n0 + s * n1)

    def fill_z(i, _):
        rows_v[0, i, :] = jnp.zeros((D,), jnp.float32)
        return 0
    lax.fori_loop(0, CHUNK, fill_z, 0)

    def zero_step(t, _):
        pltpu.sync_copy(rows_v.at[0],
                        acc_sh.at[pl.ds(s * ROWS_PER_TILE + t * CHUNK, CHUNK)])
        return 0
    lax.fori_loop(0, ROWS_PER_TILE // CHUNK, zero_step, 0)
    pltpu.sync_copy(rows_v.at[0, pl.ds(0, ROWS_PER_TILE % CHUNK)],
                    acc_sh.at[pl.ds(s * ROWS_PER_TILE
                                    + (ROWS_PER_TILE // CHUNK) * CHUNK,
                                    ROWS_PER_TILE % CHUNK)])
    plsc.subcore_barrier()

    pltpu.sync_copy(src_hbm.at[base], sidx_v.at[0])
    pltpu.sync_copy(dst_hbm.at[base], didx_v.at[0])

    @pl.when(nchunks > 1)
    def _():
        pltpu.async_copy(src_hbm.at[base + 1], sidx_v.at[1], isem)
        pltpu.async_copy(dst_hbm.at[base + 1], didx_v.at[1], isem)

    pltpu.async_copy(g_hbm.at[sidx_v.at[0]], rows_v.at[0], gsem)

    def chunk_step(j, _):
        cur = lax.rem(j, 2)
        nxt = lax.rem(j + 1, 2)
        dcur = lax.rem(j, 3)
        dnxt = lax.rem(j + 1, 3)

        @pl.when(j >= 1)
        def _():
            pltpu.make_async_copy(rows_v.at[nxt],
                                  acc_sh.at[didx_v.at[lax.rem(j + 2, 3)]],
                                  ssem).wait()

        @pl.when(j + 1 < nchunks)
        def _():
            pltpu.make_async_copy(src_hbm.at[base + j + 1], sidx_v.at[nxt], isem).wait()
            pltpu.make_async_copy(dst_hbm.at[base + j + 1], didx_v.at[dnxt], isem).wait()
            pltpu.async_copy(g_hbm.at[sidx_v.at[nxt]], rows_v.at[nxt], gsem)

        pltpu.make_async_copy(g_hbm.at[sidx_v.at[cur]], rows_v.at[cur], gsem).wait()
        pltpu.async_copy(rows_v.at[cur], acc_sh.at[didx_v.at[dcur]], ssem, add=True)

        @pl.when(j + 2 < nchunks)
        def _():
            pltpu.async_copy(src_hbm.at[base + j + 2], sidx_v.at[cur], isem)
            pltpu.async_copy(dst_hbm.at[base + j + 2], didx_v.at[lax.rem(j + 2, 3)], isem)
        return 0

    lax.fori_loop(0, nchunks, chunk_step, 0)
    pltpu.make_async_copy(rows_v.at[lax.rem(nchunks - 1, 2)],
                          acc_sh.at[didx_v.at[lax.rem(nchunks - 1, 3)]], ssem).wait()
    plsc.subcore_barrier()

    def dump_step(t, _):
        o = s * ROWS_PER_TILE + t * CHUNK
        pltpu.sync_copy(acc_sh.at[pl.ds(o, CHUNK)], out_hbm.at[c, pl.ds(o, CHUNK)])
        return 0
    lax.fori_loop(0, ROWS_PER_TILE // CHUNK, dump_step, 0)
    o = s * ROWS_PER_TILE + (ROWS_PER_TILE // CHUNK) * CHUNK
    r = ROWS_PER_TILE % CHUNK
    pltpu.sync_copy(acc_sh.at[pl.ds(o, r)], out_hbm.at[c, pl.ds(o, r)])


def _make_scatter(n0, n1):
    return functools.partial(
        pl.kernel,
        out_type=jax.ShapeDtypeStruct((2, N_PAD, D), jnp.float32),
        mesh=_mesh,
        scratch_types=[
            pltpu.VMEM((2, CHUNK), jnp.int32),
            pltpu.VMEM((3, CHUNK), jnp.int32),
            pltpu.VMEM((2, CHUNK, D), jnp.float32),
            pltpu.VMEM_SHARED((N_PAD, D), jnp.float32),
            pltpu.SemaphoreType.DMA,
            pltpu.SemaphoreType.DMA,
            pltpu.SemaphoreType.DMA,
        ],
    )(functools.partial(_scatter_body, n0, n1))


def _pre_body(x_ref, w_ref, histt_ref, g_ref):
    h = lax.dot_general(x_ref[...], w_ref[...],
                        (((1,), (1,)), ((), ())),
                        preferred_element_type=jnp.float32)
    deg = histt_ref[:, 0:1] + histt_ref[:, 1:2] + 1.0
    g_ref[...] = h * lax.rsqrt(deg)


def _post_body(pa_ref, pb_ref, g_ref, histt_ref, b_ref, out_ref):
    acc = pa_ref[...] + pb_ref[...] + g_ref[...]
    deg = histt_ref[:, 0:1] + histt_ref[:, 1:2] + 1.0
    out = acc * lax.rsqrt(deg) + b_ref[...]
    out_ref[...] = jnp.maximum(out, 0.0)


_BLK = 256
_GRID = N_PAD // _BLK


def _tc_pre(x_p, W, histt):
    return pl.pallas_call(
        _pre_body,
        grid=(_GRID,),
        in_specs=[
            pl.BlockSpec((_BLK, D), lambda i: (i, 0)),
            pl.BlockSpec((D, D), lambda i: (0, 0)),
            pl.BlockSpec((_BLK, 2), lambda i: (i, 0)),
        ],
        out_specs=pl.BlockSpec((_BLK, D), lambda i: (i, 0)),
        out_shape=jax.ShapeDtypeStruct((N_PAD, D), jnp.float32),
    )(x_p, W, histt)


def _tc_post(pa, pb, g, histt, b2d):
    return pl.pallas_call(
        _post_body,
        grid=(_GRID,),
        in_specs=[
            pl.BlockSpec((_BLK, D), lambda i: (i, 0)),
            pl.BlockSpec((_BLK, D), lambda i: (i, 0)),
            pl.BlockSpec((_BLK, D), lambda i: (i, 0)),
            pl.BlockSpec((_BLK, 2), lambda i: (i, 0)),
            pl.BlockSpec((1, D), lambda i: (0, 0)),
        ],
        out_specs=pl.BlockSpec((_BLK, D), lambda i: (i, 0)),
        out_shape=jax.ShapeDtypeStruct((N_PAD, D), jnp.float32),
    )(pa, pb, g, histt, b2d)


_SLOW_SHARE = 105 / 157


def kernel(x, edge_index, W, b):
    src = edge_index[0].astype(jnp.int32)
    dst = edge_index[1].astype(jnp.int32)
    E = src.shape[0]

    per_w = -(-E // NW)
    nch_h = -(-per_w // CHUNK)
    pad_h = NW * nch_h * CHUNK - E
    pad_dst_h = N_NODES + jnp.arange(pad_h, dtype=jnp.int32) % (N_PAD - N_NODES)
    dst_h = jnp.concatenate([dst, pad_dst_h]).reshape(NW, nch_h, CHUNK)

    per_tile = -(-(-(-E // CHUNK)) // NS)
    n0 = max(1, round(per_tile * _SLOW_SHARE))
    n1 = per_tile - n0
    e_pad = NS * per_tile * CHUNK
    pad_n = e_pad - E
    pad_dst = N_NODES + jnp.arange(pad_n, dtype=jnp.int32) % (N_PAD - N_NODES)
    src_p = jnp.pad(src, (0, pad_n)).reshape(NS * per_tile, CHUNK)
    dst_p = jnp.concatenate([dst, pad_dst]).reshape(NS * per_tile, CHUNK)

    x_p = jnp.pad(x, ((0, N_PAD - x.shape[0]), (0, 0)))

    hist = _make_hist(nch_h)(dst_h)
    histt = hist.T
    g = _tc_pre(x_p, W, histt)
    parts = _make_scatter(n0, n1)(src_p, dst_p, g)
    out = _tc_post(parts[0], parts[1], g, histt, b.reshape(1, D))
    return out[:x.shape[0]]


# speedup vs baseline: 1.2408x; 1.1775x over previous
import functools

import jax
import jax.numpy as jnp
from jax import lax
from jax.experimental import pallas as pl
from jax.experimental.pallas import tpu as pltpu
from jax.experimental.pallas import tpu_sc as plsc

N_NODES = 10000
D = 128
N_PAD = 10240
NW = 32
NS = 16
CHUNK = 128
ROWS_PER_TILE = N_PAD // NS

_mesh = plsc.VectorSubcoreMesh(core_axis_name="c", subcore_axis_name="s")


def _hist_body(totc, dst_hbm, out_hbm, idx_v, ones_v, zbuf_v, hist_sh):
    s = lax.axis_index("s")
    c = lax.axis_index("c")
    w = c * NS + s
    nb = 8 * (totc // (8 * NW))
    ka = (totc - NW * nb) // 8
    na = nb + 8
    nchunks = nb + jnp.where(w < ka, 8, 0)
    base = nb * w + 8 * lax.min(w, ka)

    def fill_ones(i, _):
        ones_v[pl.ds(i * 16, 16)] = jnp.ones((16,), jnp.float32)
        return 0
    lax.fori_loop(0, CHUNK // 16, fill_ones, 0)

    def fill_z(i, _):
        zbuf_v[pl.ds(i * 16, 16)] = jnp.zeros((16,), jnp.float32)
        return 0
    lax.fori_loop(0, ROWS_PER_TILE // 16, fill_z, 0)

    pltpu.sync_copy(zbuf_v, hist_sh.at[pl.ds(s * ROWS_PER_TILE, ROWS_PER_TILE)])
    plsc.subcore_barrier()

    @pl.when(w < ka)
    def _():
        pltpu.sync_copy(dst_hbm.at[pl.ds(base, na)], idx_v.at[pl.ds(0, na)])

    @pl.when(w >= ka)
    def _():
        pltpu.sync_copy(dst_hbm.at[pl.ds(base, nb)], idx_v.at[pl.ds(0, nb)])

    def chunk_step(j, _):
        pltpu.sync_copy(ones_v, hist_sh.at[idx_v.at[j]], add=True)
        return 0
    lax.fori_loop(0, nchunks, chunk_step, 0)
    plsc.subcore_barrier()

    pltpu.sync_copy(hist_sh.at[pl.ds(s * ROWS_PER_TILE, ROWS_PER_TILE)],
                    out_hbm.at[c, pl.ds(s * ROWS_PER_TILE, ROWS_PER_TILE)])


def _make_hist(totc):
    na = 8 * (totc // (8 * NW)) + 8
    return functools.partial(
        pl.kernel,
        out_type=jax.ShapeDtypeStruct((2, N_PAD), jnp.float32),
        mesh=_mesh,
        scratch_types=[
            pltpu.VMEM((na, CHUNK), jnp.int32),
            pltpu.VMEM((CHUNK,), jnp.float32),
            pltpu.VMEM((ROWS_PER_TILE,), jnp.float32),
            pltpu.VMEM_SHARED((N_PAD,), jnp.float32),
        ],
    )(functools.partial(_hist_body, totc))


def _scatter_body(n0, n1, src_hbm, dst_hbm, g_hbm, out_hbm,
                  sidx_v, didx_v, rows_v, acc_sh, gsem, isem, ssem):
    s = lax.axis_index("s")
    c = lax.axis_index("c")
    nchunks = lax.select(c == 0, n0, n1)
    base = lax.select(c == 0, s * n0, NS * n0 + s * n1)

    def fill_z(i, _):
        rows_v[0, i, :] = jnp.zeros((D,), jnp.float32)
        return 0
    lax.fori_loop(0, CHUNK, fill_z, 0)

    def zero_step(t, _):
        pltpu.sync_copy(rows_v.at[0],
                        acc_sh.at[pl.ds(s * ROWS_PER_TILE + t * CHUNK, CHUNK)])
        return 0
    lax.fori_loop(0, ROWS_PER_TILE // CHUNK, zero_step, 0)
    pltpu.sync_copy(rows_v.at[0, pl.ds(0, ROWS_PER_TILE % CHUNK)],
                    acc_sh.at[pl.ds(s * ROWS_PER_TILE
                                    + (ROWS_PER_TILE // CHUNK) * CHUNK,
                                    ROWS_PER_TILE % CHUNK)])
    plsc.subcore_barrier()

    pltpu.sync_copy(src_hbm.at[base], sidx_v.at[0])
    pltpu.sync_copy(dst_hbm.at[base], didx_v.at[0])

    @pl.when(nchunks > 1)
    def _():
        pltpu.async_copy(src_hbm.at[base + 1], sidx_v.at[1], isem)
        pltpu.async_copy(dst_hbm.at[base + 1], didx_v.at[1], isem)

    pltpu.async_copy(g_hbm.at[sidx_v.at[0]], rows_v.at[0], gsem)

    def chunk_step(j, _):
        cur = lax.rem(j, 2)
        nxt = lax.rem(j + 1, 2)
        dcur = lax.rem(j, 3)
        dnxt = lax.rem(j + 1, 3)

        @pl.when(j >= 1)
        def _():
            pltpu.make_async_copy(rows_v.at[nxt],
                                  acc_sh.at[didx_v.at[lax.rem(j + 2, 3)]],
                                  ssem).wait()

        @pl.when(j + 1 < nchunks)
        def _():
            pltpu.make_async_copy(src_hbm.at[base + j + 1], sidx_v.at[nxt], isem).wait()
            pltpu.make_async_copy(dst_hbm.at[base + j + 1], didx_v.at[dnxt], isem).wait()
            pltpu.async_copy(g_hbm.at[sidx_v.at[nxt]], rows_v.at[nxt], gsem)

        pltpu.make_async_copy(g_hbm.at[sidx_v.at[cur]], rows_v.at[cur], gsem).wait()
        pltpu.async_copy(rows_v.at[cur], acc_sh.at[didx_v.at[dcur]], ssem, add=True)

        @pl.when(j + 2 < nchunks)
        def _():
            pltpu.async_copy(src_hbm.at[base + j + 2], sidx_v.at[cur], isem)
            pltpu.async_copy(dst_hbm.at[base + j + 2], didx_v.at[lax.rem(j + 2, 3)], isem)
        return 0

    lax.fori_loop(0, nchunks, chunk_step, 0)
    pltpu.make_async_copy(rows_v.at[lax.rem(nchunks - 1, 2)],
                          acc_sh.at[didx_v.at[lax.rem(nchunks - 1, 3)]], ssem).wait()
    plsc.subcore_barrier()

    def dump_step(t, _):
        o = s * ROWS_PER_TILE + t * CHUNK
        pltpu.sync_copy(acc_sh.at[pl.ds(o, CHUNK)], out_hbm.at[c, pl.ds(o, CHUNK)])
        return 0
    lax.fori_loop(0, ROWS_PER_TILE // CHUNK, dump_step, 0)
    o = s * ROWS_PER_TILE + (ROWS_PER_TILE // CHUNK) * CHUNK
    r = ROWS_PER_TILE % CHUNK
    pltpu.sync_copy(acc_sh.at[pl.ds(o, r)], out_hbm.at[c, pl.ds(o, r)])


def _make_scatter(n0, n1):
    return functools.partial(
        pl.kernel,
        out_type=jax.ShapeDtypeStruct((2, N_PAD, D), jnp.float32),
        mesh=_mesh,
        scratch_types=[
            pltpu.VMEM((2, CHUNK), jnp.int32),
            pltpu.VMEM((3, CHUNK), jnp.int32),
            pltpu.VMEM((2, CHUNK, D), jnp.float32),
            pltpu.VMEM_SHARED((N_PAD, D), jnp.float32),
            pltpu.SemaphoreType.DMA,
            pltpu.SemaphoreType.DMA,
            pltpu.SemaphoreType.DMA,
        ],
    )(functools.partial(_scatter_body, n0, n1))


def _pre_body(x_ref, w_ref, histt_ref, g_ref):
    h = lax.dot_general(x_ref[...], w_ref[...],
                        (((1,), (1,)), ((), ())),
                        preferred_element_type=jnp.float32)
    deg = histt_ref[:, 0:1] + histt_ref[:, 1:2] + 1.0
    g_ref[...] = h * lax.rsqrt(deg)


_BLK = 1000
_GRID = N_NODES // _BLK


def _tc_pre(x, W, histt):
    return pl.pallas_call(
        _pre_body,
        grid=(_GRID,),
        in_specs=[
            pl.BlockSpec((_BLK, D), lambda i: (i, 0)),
            pl.BlockSpec((D, D), lambda i: (0, 0)),
            pl.BlockSpec((_BLK, 2), lambda i: (i, 0)),
        ],
        out_specs=pl.BlockSpec((_BLK, D), lambda i: (i, 0)),
        out_shape=jax.ShapeDtypeStruct((N_PAD, D), jnp.float32),
    )(x, W, histt)


def _post_body(parts_ref, g_ref, histt_ref, b_ref, out_ref):
    acc = parts_ref[0] + parts_ref[1] + g_ref[...]
    deg = histt_ref[:, 0:1] + histt_ref[:, 1:2] + 1.0
    out = acc * lax.rsqrt(deg) + b_ref[...]
    out_ref[...] = jnp.maximum(out, 0.0)


def _tc_post(parts, g, histt, b2d):
    return pl.pallas_call(
        _post_body,
        grid=(_GRID,),
        in_specs=[
            pl.BlockSpec((2, _BLK, D), lambda i: (0, i, 0)),
            pl.BlockSpec((_BLK, D), lambda i: (i, 0)),
            pl.BlockSpec((_BLK, 2), lambda i: (i, 0)),
            pl.BlockSpec((1, D), lambda i: (0, 0)),
        ],
        out_specs=pl.BlockSpec((_BLK, D), lambda i: (i, 0)),
        out_shape=jax.ShapeDtypeStruct((N_NODES, D), jnp.float32),
    )(parts, g, histt, b2d)


_SLOW_SHARE = 105 / 157


def kernel(x, edge_index, W, b):
    src = edge_index[0].astype(jnp.int32)
    dst = edge_index[1].astype(jnp.int32)
    E = src.shape[0]

    per_tile = -(-(-(-E // CHUNK)) // NS)
    n0 = max(1, round(per_tile * _SLOW_SHARE))
    n1 = per_tile - n0
    totc = NS * per_tile
    pad_n = (totc + 1) * CHUNK - E
    pad_dst = N_NODES + jnp.arange(pad_n, dtype=jnp.int32) % (N_PAD - N_NODES)
    src_p = jnp.pad(src, (0, pad_n)).reshape(totc + 1, CHUNK)
    dst_p = jnp.concatenate([dst, pad_dst]).reshape(totc + 1, CHUNK)

    hist = _make_hist(totc)(dst_p)
    histt = hist.T
    g = _tc_pre(x, W, histt)
    parts = _make_scatter(n0, n1)(src_p, dst_p, g)
    return _tc_post(parts, g, histt, b.reshape(1, D))


# speedup vs baseline: 1.3977x; 1.1265x over previous
import functools

import jax
import jax.numpy as jnp
from jax import lax
from jax.experimental import pallas as pl
from jax.experimental.pallas import tpu as pltpu
from jax.experimental.pallas import tpu_sc as plsc

N_NODES = 10000
D = 128
N_PAD = 10240
NW = 32
NS = 16
CHUNK = 128
ROWS_PER_TILE = N_PAD // NS

_mesh = plsc.VectorSubcoreMesh(core_axis_name="c", subcore_axis_name="s")


def _hist_body(totc, ei_hbm, out_hbm, idx_v, ones_v, zbuf_v, hist_sh):
    s = lax.axis_index("s")
    c = lax.axis_index("c")
    w = c * NS + s
    nb = 8 * (totc // (8 * NW))
    left = totc - NW * nb
    ka = left // 8
    rem8 = left % 8
    na = nb + 8
    nchunks = (nb + jnp.where(w < ka, 8, 0)
               + jnp.where(w == NW - 1, rem8, 0))
    base = nb * w + 8 * jnp.minimum(w, ka)

    def fill_ones(i, _):
        ones_v[pl.ds(i * 16, 16)] = jnp.ones((16,), jnp.float32)
        return 0
    lax.fori_loop(0, CHUNK // 16, fill_ones, 0)

    def fill_z(i, _):
        zbuf_v[pl.ds(i * 16, 16)] = jnp.zeros((16,), jnp.float32)
        return 0
    lax.fori_loop(0, ROWS_PER_TILE // 16, fill_z, 0)

    pltpu.sync_copy(zbuf_v, hist_sh.at[pl.ds(s * ROWS_PER_TILE, ROWS_PER_TILE)])
    plsc.subcore_barrier()

    @pl.when(w < ka)
    def _():
        pltpu.sync_copy(ei_hbm.at[1, pl.ds(base, na)], idx_v.at[pl.ds(0, na)])

    @pl.when(jnp.logical_and(w >= ka, w < NW - 1))
    def _():
        pltpu.sync_copy(ei_hbm.at[1, pl.ds(base, nb)], idx_v.at[pl.ds(0, nb)])

    @pl.when(w == NW - 1)
    def _():
        pltpu.sync_copy(ei_hbm.at[1, pl.ds(base, nb)], idx_v.at[pl.ds(0, nb)])
        for i in range(rem8):
            pltpu.sync_copy(ei_hbm.at[1, base + nb + i], idx_v.at[nb + i])

    def chunk_step(j, _):
        pltpu.sync_copy(ones_v, hist_sh.at[idx_v.at[j]], add=True)
        return 0
    lax.fori_loop(0, nchunks, chunk_step, 0)
    plsc.subcore_barrier()

    pltpu.sync_copy(hist_sh.at[pl.ds(s * ROWS_PER_TILE, ROWS_PER_TILE)],
                    out_hbm.at[c, pl.ds(s * ROWS_PER_TILE, ROWS_PER_TILE)])


def _make_hist(totc):
    na = 8 * (totc // (8 * NW)) + 8
    return functools.partial(
        pl.kernel,
        out_type=jax.ShapeDtypeStruct((2, N_PAD), jnp.float32),
        mesh=_mesh,
        scratch_types=[
            pltpu.VMEM((na, CHUNK), jnp.int32),
            pltpu.VMEM((CHUNK,), jnp.float32),
            pltpu.VMEM((ROWS_PER_TILE,), jnp.float32),
            pltpu.VMEM_SHARED((N_PAD,), jnp.float32),
        ],
    )(functools.partial(_hist_body, totc))


def _scatter_body(n0, n1, rem, ei_hbm, g_hbm, out_hbm,
                  sidx_v, didx_v, rows_v, acc_sh, gsem, isem, ssem):
    s = lax.axis_index("s")
    c = lax.axis_index("c")
    nchunks = jnp.where(c == 0, n0 + jnp.where(s < rem, 1, 0), n1)
    base = jnp.where(c == 0, s * n0 + jnp.minimum(s, rem),
                     NS * n0 + rem + s * n1)

    def fill_z(i, _):
        rows_v[0, i, :] = jnp.zeros((D,), jnp.float32)
        return 0
    lax.fori_loop(0, CHUNK, fill_z, 0)

    def zero_step(t, _):
        pltpu.sync_copy(rows_v.at[0],
                        acc_sh.at[pl.ds(s * ROWS_PER_TILE + t * CHUNK, CHUNK)])
        return 0
    lax.fori_loop(0, ROWS_PER_TILE // CHUNK, zero_step, 0)
    pltpu.sync_copy(rows_v.at[0, pl.ds(0, ROWS_PER_TILE % CHUNK)],
                    acc_sh.at[pl.ds(s * ROWS_PER_TILE
                                    + (ROWS_PER_TILE // CHUNK) * CHUNK,
                                    ROWS_PER_TILE % CHUNK)])
    plsc.subcore_barrier()

    pltpu.sync_copy(ei_hbm.at[0, base], sidx_v.at[0])
    pltpu.sync_copy(ei_hbm.at[1, base], didx_v.at[0])

    @pl.when(nchunks > 1)
    def _():
        pltpu.async_copy(ei_hbm.at[0, base + 1], sidx_v.at[1], isem)
        pltpu.async_copy(ei_hbm.at[1, base + 1], didx_v.at[1], isem)

    pltpu.async_copy(g_hbm.at[sidx_v.at[0]], rows_v.at[0], gsem)

    def chunk_step(j, _):
        cur = lax.rem(j, 2)
        nxt = lax.rem(j + 1, 2)
        dcur = lax.rem(j, 3)
        dnxt = lax.rem(j + 1, 3)

        @pl.when(j >= 1)
        def _():
            pltpu.make_async_copy(rows_v.at[nxt],
                                  acc_sh.at[didx_v.at[lax.rem(j + 2, 3)]],
                                  ssem).wait()

        @pl.when(j + 1 < nchunks)
        def _():
            pltpu.make_async_copy(ei_hbm.at[0, base + j + 1], sidx_v.at[nxt], isem).wait()
            pltpu.make_async_copy(ei_hbm.at[1, base + j + 1], didx_v.at[dnxt], isem).wait()
            pltpu.async_copy(g_hbm.at[sidx_v.at[nxt]], rows_v.at[nxt], gsem)

        pltpu.make_async_copy(g_hbm.at[sidx_v.at[cur]], rows_v.at[cur], gsem).wait()
        pltpu.async_copy(rows_v.at[cur], acc_sh.at[didx_v.at[dcur]], ssem, add=True)

        @pl.when(j + 2 < nchunks)
        def _():
            pltpu.async_copy(ei_hbm.at[0, base + j + 2], sidx_v.at[cur], isem)
            pltpu.async_copy(ei_hbm.at[1, base + j + 2], didx_v.at[lax.rem(j + 2, 3)], isem)
        return 0

    lax.fori_loop(0, nchunks, chunk_step, 0)
    pltpu.make_async_copy(rows_v.at[lax.rem(nchunks - 1, 2)],
                          acc_sh.at[didx_v.at[lax.rem(nchunks - 1, 3)]], ssem).wait()
    plsc.subcore_barrier()

    def dump_step(t, _):
        o = s * ROWS_PER_TILE + t * CHUNK
        pltpu.sync_copy(acc_sh.at[pl.ds(o, CHUNK)], out_hbm.at[c, pl.ds(o, CHUNK)])
        return 0
    lax.fori_loop(0, ROWS_PER_TILE // CHUNK, dump_step, 0)
    o = s * ROWS_PER_TILE + (ROWS_PER_TILE // CHUNK) * CHUNK
    r = ROWS_PER_TILE % CHUNK
    pltpu.sync_copy(acc_sh.at[pl.ds(o, r)], out_hbm.at[c, pl.ds(o, r)])


def _make_scatter(n0, n1, rem):
    return functools.partial(
        pl.kernel,
        out_type=jax.ShapeDtypeStruct((2, N_PAD, D), jnp.float32),
        mesh=_mesh,
        scratch_types=[
            pltpu.VMEM((2, CHUNK), jnp.int32),
            pltpu.VMEM((3, CHUNK), jnp.int32),
            pltpu.VMEM((2, CHUNK, D), jnp.float32),
            pltpu.VMEM_SHARED((N_PAD, D), jnp.float32),
            pltpu.SemaphoreType.DMA,
            pltpu.SemaphoreType.DMA,
            pltpu.SemaphoreType.DMA,
        ],
    )(functools.partial(_scatter_body, n0, n1, rem))


def _pre_body(x_ref, w_ref, histt_ref, g_ref):
    h = lax.dot_general(x_ref[...], w_ref[...],
                        (((1,), (1,)), ((), ())),
                        preferred_element_type=jnp.float32)
    deg = histt_ref[:, 0:1] + histt_ref[:, 1:2] + 1.0
    g_ref[...] = h * lax.rsqrt(deg)


_BLK = 1000
_GRID = N_NODES // _BLK


def _tc_pre(x, W, histt):
    return pl.pallas_call(
        _pre_body,
        grid=(_GRID,),
        in_specs=[
            pl.BlockSpec((_BLK, D), lambda i: (i, 0)),
            pl.BlockSpec((D, D), lambda i: (0, 0)),
            pl.BlockSpec((_BLK, 2), lambda i: (i, 0)),
        ],
        out_specs=pl.BlockSpec((_BLK, D), lambda i: (i, 0)),
        out_shape=jax.ShapeDtypeStruct((N_PAD, D), jnp.float32),
    )(x, W, histt)


def _post_body(parts_ref, g_ref, histt_ref, b_ref, out_ref):
    acc = parts_ref[0] + parts_ref[1] + g_ref[...]
    deg = histt_ref[:, 0:1] + histt_ref[:, 1:2] + 1.0
    out = acc * lax.rsqrt(deg) + b_ref[...]
    out_ref[...] = jnp.maximum(out, 0.0)


def _tc_post(parts, g, histt, b2d):
    return pl.pallas_call(
        _post_body,
        grid=(_GRID,),
        in_specs=[
            pl.BlockSpec((2, _BLK, D), lambda i: (0, i, 0)),
            pl.BlockSpec((_BLK, D), lambda i: (i, 0)),
            pl.BlockSpec((_BLK, 2), lambda i: (i, 0)),
            pl.BlockSpec((1, D), lambda i: (0, 0)),
        ],
        out_specs=pl.BlockSpec((_BLK, D), lambda i: (i, 0)),
        out_shape=jax.ShapeDtypeStruct((N_NODES, D), jnp.float32),
    )(parts, g, histt, b2d)


_SLOW_SHARE = 105 / 157


def kernel(x, edge_index, W, b):
    ei = edge_index.astype(jnp.int32)
    E = ei.shape[1]

    if E % CHUNK:
        pad_n = CHUNK - E % CHUNK
        pad_dst = N_NODES + jnp.arange(pad_n, dtype=jnp.int32) % (N_PAD - N_NODES)
        ei = jnp.concatenate(
            [ei, jnp.stack([jnp.zeros((pad_n,), jnp.int32), pad_dst])], axis=1)

    totc = ei.shape[1] // CHUNK
    ei3 = ei.reshape(2, totc, CHUNK)
    per_tile = totc // NS
    rem = totc - NS * per_tile
    n0 = max(1, round(per_tile * _SLOW_SHARE))
    n1 = per_tile - n0

    hist = _make_hist(totc)(ei3)
    histt = hist.T
    g = _tc_pre(x, W, histt)
    parts = _make_scatter(n0, n1, rem)(ei3, g)
    return _tc_post(parts, g, histt, b.reshape(1, D))


# speedup vs baseline: 1.6119x; 1.1533x over previous
import functools

import jax
import jax.numpy as jnp
from jax import lax
from jax.experimental import pallas as pl
from jax.experimental.pallas import tpu as pltpu
from jax.experimental.pallas import tpu_sc as plsc

N_NODES = 10000
D = 128
N_PAD = 10240
NW = 32
NS = 16
CHUNK = 128
ROWS_PER_TILE = N_PAD // NS

_mesh = plsc.VectorSubcoreMesh(core_axis_name="c", subcore_axis_name="s")


def _hist_body(totc, ei_hbm, out_hbm, idx_v, ones_v, zbuf_v, hist_sh):
    s = lax.axis_index("s")
    c = lax.axis_index("c")
    w = c * NS + s
    nb = 8 * (totc // (8 * NW))
    left = totc - NW * nb
    ka = left // 8
    rem8 = left % 8
    na = nb + 8
    nchunks = (nb + jnp.where(w < ka, 8, 0)
               + jnp.where(w == NW - 1, rem8, 0))
    base = nb * w + 8 * jnp.minimum(w, ka)

    def fill_ones(i, _):
        ones_v[pl.ds(i * 16, 16)] = jnp.ones((16,), jnp.float32)
        return 0
    lax.fori_loop(0, CHUNK // 16, fill_ones, 0)

    def fill_z(i, _):
        zbuf_v[pl.ds(i * 16, 16)] = jnp.zeros((16,), jnp.float32)
        return 0
    lax.fori_loop(0, ROWS_PER_TILE // 16, fill_z, 0)

    pltpu.sync_copy(zbuf_v, hist_sh.at[pl.ds(s * ROWS_PER_TILE, ROWS_PER_TILE)])
    plsc.subcore_barrier()

    @pl.when(w < ka)
    def _():
        pltpu.sync_copy(ei_hbm.at[1, pl.ds(base, na)], idx_v.at[pl.ds(0, na)])

    @pl.when(jnp.logical_and(w >= ka, w < NW - 1))
    def _():
        pltpu.sync_copy(ei_hbm.at[1, pl.ds(base, nb)], idx_v.at[pl.ds(0, nb)])

    @pl.when(w == NW - 1)
    def _():
        pltpu.sync_copy(ei_hbm.at[1, pl.ds(base, nb)], idx_v.at[pl.ds(0, nb)])
        for i in range(rem8):
            pltpu.sync_copy(ei_hbm.at[1, base + nb + i], idx_v.at[nb + i])

    def chunk_step(j, _):
        pltpu.sync_copy(ones_v, hist_sh.at[idx_v.at[j]], add=True)
        return 0
    lax.fori_loop(0, nchunks, chunk_step, 0)
    plsc.subcore_barrier()

    pltpu.sync_copy(hist_sh.at[pl.ds(s * ROWS_PER_TILE, ROWS_PER_TILE)],
                    out_hbm.at[c, pl.ds(s * ROWS_PER_TILE, ROWS_PER_TILE)])


def _make_hist(totc):
    na = 8 * (totc // (8 * NW)) + 8
    return functools.partial(
        pl.kernel,
        out_type=jax.ShapeDtypeStruct((2, N_PAD), jnp.float32),
        mesh=_mesh,
        scratch_types=[
            pltpu.VMEM((na, CHUNK), jnp.int32),
            pltpu.VMEM((CHUNK,), jnp.float32),
            pltpu.VMEM((ROWS_PER_TILE,), jnp.float32),
            pltpu.VMEM_SHARED((N_PAD,), jnp.float32),
        ],
    )(functools.partial(_hist_body, totc))


def _scatter_body(n0, n1, rem, ei_hbm, g_hbm, out_hbm,
                  sidx_v, didx_v, rows_v, acc_sh, gsem, isem, ssem):
    s = lax.axis_index("s")
    c = lax.axis_index("c")
    nchunks = jnp.where(c == 0, n0 + jnp.where(s < rem, 1, 0), n1)
    base = jnp.where(c == 0, s * n0 + jnp.minimum(s, rem),
                     NS * n0 + rem + s * n1)

    def fill_z(i, _):
        rows_v[0, i, :] = jnp.zeros((D,), jnp.float32)
        return 0
    lax.fori_loop(0, CHUNK, fill_z, 0)

    def zero_step(t, _):
        pltpu.sync_copy(rows_v.at[0],
                        acc_sh.at[pl.ds(s * ROWS_PER_TILE + t * CHUNK, CHUNK)])
        return 0
    lax.fori_loop(0, ROWS_PER_TILE // CHUNK, zero_step, 0)
    pltpu.sync_copy(rows_v.at[0, pl.ds(0, ROWS_PER_TILE % CHUNK)],
                    acc_sh.at[pl.ds(s * ROWS_PER_TILE
                                    + (ROWS_PER_TILE // CHUNK) * CHUNK,
                                    ROWS_PER_TILE % CHUNK)])
    plsc.subcore_barrier()

    pltpu.sync_copy(ei_hbm.at[0, base], sidx_v.at[0])
    pltpu.sync_copy(ei_hbm.at[1, base], didx_v.at[0])

    @pl.when(nchunks > 1)
    def _():
        pltpu.async_copy(ei_hbm.at[0, base + 1], sidx_v.at[1], isem)
        pltpu.async_copy(ei_hbm.at[1, base + 1], didx_v.at[1], isem)

    pltpu.async_copy(g_hbm.at[sidx_v.at[0]], rows_v.at[0], gsem)

    def chunk_step(j, _):
        cur = lax.rem(j, 2)
        nxt = lax.rem(j + 1, 2)
        dcur = lax.rem(j, 3)
        dnxt = lax.rem(j + 1, 3)

        @pl.when(j >= 1)
        def _():
            pltpu.make_async_copy(rows_v.at[nxt],
                                  acc_sh.at[didx_v.at[lax.rem(j + 2, 3)]],
                                  ssem).wait()

        @pl.when(j + 1 < nchunks)
        def _():
            pltpu.make_async_copy(ei_hbm.at[0, base + j + 1], sidx_v.at[nxt], isem).wait()
            pltpu.make_async_copy(ei_hbm.at[1, base + j + 1], didx_v.at[dnxt], isem).wait()
            pltpu.async_copy(g_hbm.at[sidx_v.at[nxt]], rows_v.at[nxt], gsem)

        pltpu.make_async_copy(g_hbm.at[sidx_v.at[cur]], rows_v.at[cur], gsem).wait()
        pltpu.async_copy(rows_v.at[cur], acc_sh.at[didx_v.at[dcur]], ssem, add=True)

        @pl.when(j + 2 < nchunks)
        def _():
            pltpu.async_copy(ei_hbm.at[0, base + j + 2], sidx_v.at[cur], isem)
            pltpu.async_copy(ei_hbm.at[1, base + j + 2], didx_v.at[lax.rem(j + 2, 3)], isem)
        return 0

    lax.fori_loop(0, nchunks, chunk_step, 0)
    pltpu.make_async_copy(rows_v.at[lax.rem(nchunks - 1, 2)],
                          acc_sh.at[didx_v.at[lax.rem(nchunks - 1, 3)]], ssem).wait()
    plsc.subcore_barrier()

    def dump_step(t, _):
        o = s * ROWS_PER_TILE + t * CHUNK
        pltpu.sync_copy(acc_sh.at[pl.ds(o, CHUNK)], out_hbm.at[c, pl.ds(o, CHUNK)])
        return 0
    lax.fori_loop(0, ROWS_PER_TILE // CHUNK, dump_step, 0)
    o = s * ROWS_PER_TILE + (ROWS_PER_TILE // CHUNK) * CHUNK
    r = ROWS_PER_TILE % CHUNK
    pltpu.sync_copy(acc_sh.at[pl.ds(o, r)], out_hbm.at[c, pl.ds(o, r)])


def _make_scatter(n0, n1, rem):
    return functools.partial(
        pl.kernel,
        out_type=jax.ShapeDtypeStruct((2, N_PAD, D), jnp.float32),
        mesh=_mesh,
        scratch_types=[
            pltpu.VMEM((2, CHUNK), jnp.int32),
            pltpu.VMEM((3, CHUNK), jnp.int32),
            pltpu.VMEM((2, CHUNK, D), jnp.float32),
            pltpu.VMEM_SHARED((N_PAD, D), jnp.float32),
            pltpu.SemaphoreType.DMA,
            pltpu.SemaphoreType.DMA,
            pltpu.SemaphoreType.DMA,
        ],
    )(functools.partial(_scatter_body, n0, n1, rem))


def _pre_body(x_ref, w_ref, histt_ref, g_ref):
    h = lax.dot_general(x_ref[...], w_ref[...],
                        (((1,), (1,)), ((), ())),
                        preferred_element_type=jnp.float32)
    deg = histt_ref[:, 0:1] + histt_ref[:, 1:2] + 1.0
    g_ref[...] = h * lax.rsqrt(deg)


_BLK = 1000
_GRID = N_NODES // _BLK


def _tc_pre(x, W, histt):
    return pl.pallas_call(
        _pre_body,
        grid=(_GRID,),
        in_specs=[
            pl.BlockSpec((_BLK, D), lambda i: (i, 0)),
            pl.BlockSpec((D, D), lambda i: (0, 0)),
            pl.BlockSpec((_BLK, 2), lambda i: (i, 0)),
        ],
        out_specs=pl.BlockSpec((_BLK, D), lambda i: (i, 0)),
        out_shape=jax.ShapeDtypeStruct((N_PAD, D), jnp.float32),
    )(x, W, histt)


def _post_body(parts_ref, g_ref, histt_ref, b_ref, out_ref):
    acc = parts_ref[0] + parts_ref[1] + g_ref[...]
    deg = histt_ref[:, 0:1] + histt_ref[:, 1:2] + 1.0
    out = acc * lax.rsqrt(deg) + b_ref[...]
    out_ref[...] = jnp.maximum(out, 0.0)


def _tc_post(parts, g, histt, b2d):
    return pl.pallas_call(
        _post_body,
        grid=(_GRID,),
        in_specs=[
            pl.BlockSpec((2, _BLK, D), lambda i: (0, i, 0)),
            pl.BlockSpec((_BLK, D), lambda i: (i, 0)),
            pl.BlockSpec((_BLK, 2), lambda i: (i, 0)),
            pl.BlockSpec((1, D), lambda i: (0, 0)),
        ],
        out_specs=pl.BlockSpec((_BLK, D), lambda i: (i, 0)),
        out_shape=jax.ShapeDtypeStruct((N_NODES, D), jnp.float32),
    )(parts, g, histt, b2d)


_SLOW_SHARE = 82 / 157


def kernel(x, edge_index, W, b):
    ei = edge_index.astype(jnp.int32)
    E = ei.shape[1]

    if E % CHUNK:
        pad_n = CHUNK - E % CHUNK
        pad_dst = N_NODES + jnp.arange(pad_n, dtype=jnp.int32) % (N_PAD - N_NODES)
        ei = jnp.concatenate(
            [ei, jnp.stack([jnp.zeros((pad_n,), jnp.int32), pad_dst])], axis=1)

    totc = ei.shape[1] // CHUNK
    ei3 = ei.reshape(2, totc, CHUNK)
    per_tile = totc // NS
    rem = totc - NS * per_tile
    n0 = max(1, round(per_tile * _SLOW_SHARE))
    n1 = per_tile - n0

    hist = _make_hist(totc)(ei3)
    histt = hist.T
    g = _tc_pre(x, W, histt)
    parts = _make_scatter(n0, n1, rem)(ei3, g)
    return _tc_post(parts, g, histt, b.reshape(1, D))


# speedup vs baseline: 1.6388x; 1.0167x over previous
import functools

import jax
import jax.numpy as jnp
from jax import lax
from jax.experimental import pallas as pl
from jax.experimental.pallas import tpu as pltpu
from jax.experimental.pallas import tpu_sc as plsc

N_NODES = 10000
D = 128
N_PAD = 10240
NW = 32
NS = 16
CHUNK = 128
ROWS_PER_TILE = N_PAD // NS

_mesh = plsc.VectorSubcoreMesh(core_axis_name="c", subcore_axis_name="s")


def _hist_body(totc, ei_hbm, out_hbm, idx_v, ones_v, zbuf_v, hist_sh):
    s = lax.axis_index("s")
    c = lax.axis_index("c")
    w = c * NS + s
    nb = 8 * (totc // (8 * NW))
    left = totc - NW * nb
    ka = left // 8
    rem8 = left % 8
    na = nb + 8
    nchunks = (nb + jnp.where(w < ka, 8, 0)
               + jnp.where(w == NW - 1, rem8, 0))
    base = nb * w + 8 * jnp.minimum(w, ka)

    def fill_ones(i, _):
        ones_v[pl.ds(i * 16, 16)] = jnp.ones((16,), jnp.float32)
        return 0
    lax.fori_loop(0, CHUNK // 16, fill_ones, 0)

    def fill_z(i, _):
        zbuf_v[pl.ds(i * 16, 16)] = jnp.zeros((16,), jnp.float32)
        return 0
    lax.fori_loop(0, ROWS_PER_TILE // 16, fill_z, 0)

    pltpu.sync_copy(zbuf_v, hist_sh.at[pl.ds(s * ROWS_PER_TILE, ROWS_PER_TILE)])
    plsc.subcore_barrier()

    @pl.when(w < ka)
    def _():
        pltpu.sync_copy(ei_hbm.at[1, pl.ds(base, na)], idx_v.at[pl.ds(0, na)])

    @pl.when(jnp.logical_and(w >= ka, w < NW - 1))
    def _():
        pltpu.sync_copy(ei_hbm.at[1, pl.ds(base, nb)], idx_v.at[pl.ds(0, nb)])

    @pl.when(w == NW - 1)
    def _():
        pltpu.sync_copy(ei_hbm.at[1, pl.ds(base, nb)], idx_v.at[pl.ds(0, nb)])
        for i in range(rem8):
            pltpu.sync_copy(ei_hbm.at[1, base + nb + i], idx_v.at[nb + i])

    def chunk_step(j, _):
        pltpu.sync_copy(ones_v, hist_sh.at[idx_v.at[j]], add=True)
        return 0
    lax.fori_loop(0, nchunks, chunk_step, 0)
    plsc.subcore_barrier()

    pltpu.sync_copy(hist_sh.at[pl.ds(s * ROWS_PER_TILE, ROWS_PER_TILE)],
                    out_hbm.at[c, pl.ds(s * ROWS_PER_TILE, ROWS_PER_TILE)])


def _make_hist(totc):
    na = 8 * (totc // (8 * NW)) + 8
    return functools.partial(
        pl.kernel,
        out_type=jax.ShapeDtypeStruct((2, N_PAD), jnp.float32),
        mesh=_mesh,
        scratch_types=[
            pltpu.VMEM((na, CHUNK), jnp.int32),
            pltpu.VMEM((CHUNK,), jnp.float32),
            pltpu.VMEM((ROWS_PER_TILE,), jnp.float32),
            pltpu.VMEM_SHARED((N_PAD,), jnp.float32),
        ],
    )(functools.partial(_hist_body, totc))


def _scatter_body(n0, n1, rem, ei_hbm, g_hbm, out_hbm,
                  sidx_v, didx_v, rows_v, acc_sh, gsem, isem, ssem):
    s = lax.axis_index("s")
    c = lax.axis_index("c")
    nchunks = jnp.where(c == 0, n0 + jnp.where(s < rem, 1, 0), n1)
    base = jnp.where(c == 0, s * n0 + jnp.minimum(s, rem),
                     NS * n0 + rem + s * n1)

    def fill_z(i, _):
        rows_v[0, i, :] = jnp.zeros((D,), jnp.float32)
        return 0
    lax.fori_loop(0, CHUNK, fill_z, 0)

    def zero_step(t, _):
        pltpu.sync_copy(rows_v.at[0],
                        acc_sh.at[pl.ds(s * ROWS_PER_TILE + t * CHUNK, CHUNK)])
        return 0
    lax.fori_loop(0, ROWS_PER_TILE // CHUNK, zero_step, 0)
    pltpu.sync_copy(rows_v.at[0, pl.ds(0, ROWS_PER_TILE % CHUNK)],
                    acc_sh.at[pl.ds(s * ROWS_PER_TILE
                                    + (ROWS_PER_TILE // CHUNK) * CHUNK,
                                    ROWS_PER_TILE % CHUNK)])
    plsc.subcore_barrier()

    pltpu.sync_copy(ei_hbm.at[0, base], sidx_v.at[0])
    pltpu.sync_copy(ei_hbm.at[1, base], didx_v.at[0])

    @pl.when(nchunks > 1)
    def _():
        pltpu.async_copy(ei_hbm.at[0, base + 1], sidx_v.at[1], isem)
        pltpu.async_copy(ei_hbm.at[1, base + 1], didx_v.at[1], isem)

    pltpu.async_copy(g_hbm.at[sidx_v.at[0]], rows_v.at[0], gsem)

    def chunk_step(j, _):
        cur = lax.rem(j, 2)
        nxt = lax.rem(j + 1, 2)
        dcur = lax.rem(j, 3)
        dnxt = lax.rem(j + 1, 3)

        @pl.when(j >= 1)
        def _():
            pltpu.make_async_copy(rows_v.at[nxt],
                                  acc_sh.at[didx_v.at[lax.rem(j + 2, 3)]],
                                  ssem).wait()

        @pl.when(j + 1 < nchunks)
        def _():
            pltpu.make_async_copy(ei_hbm.at[0, base + j + 1], sidx_v.at[nxt], isem).wait()
            pltpu.make_async_copy(ei_hbm.at[1, base + j + 1], didx_v.at[dnxt], isem).wait()
            pltpu.async_copy(g_hbm.at[sidx_v.at[nxt]], rows_v.at[nxt], gsem)

        pltpu.make_async_copy(g_hbm.at[sidx_v.at[cur]], rows_v.at[cur], gsem).wait()
        pltpu.async_copy(rows_v.at[cur], acc_sh.at[didx_v.at[dcur]], ssem, add=True)

        @pl.when(j + 2 < nchunks)
        def _():
            pltpu.async_copy(ei_hbm.at[0, base + j + 2], sidx_v.at[cur], isem)
            pltpu.async_copy(ei_hbm.at[1, base + j + 2], didx_v.at[lax.rem(j + 2, 3)], isem)
        return 0

    lax.fori_loop(0, nchunks, chunk_step, 0)
    pltpu.make_async_copy(rows_v.at[lax.rem(nchunks - 1, 2)],
                          acc_sh.at[didx_v.at[lax.rem(nchunks - 1, 3)]], ssem).wait()
    plsc.subcore_barrier()

    def dump_step(t, _):
        o = s * ROWS_PER_TILE + t * CHUNK
        pltpu.sync_copy(acc_sh.at[pl.ds(o, CHUNK)], out_hbm.at[c, pl.ds(o, CHUNK)])
        return 0
    lax.fori_loop(0, ROWS_PER_TILE // CHUNK, dump_step, 0)
    o = s * ROWS_PER_TILE + (ROWS_PER_TILE // CHUNK) * CHUNK
    r = ROWS_PER_TILE % CHUNK
    pltpu.sync_copy(acc_sh.at[pl.ds(o, r)], out_hbm.at[c, pl.ds(o, r)])


def _make_scatter(n0, n1, rem):
    return functools.partial(
        pl.kernel,
        out_type=jax.ShapeDtypeStruct((2, N_PAD, D), jnp.float32),
        mesh=_mesh,
        scratch_types=[
            pltpu.VMEM((2, CHUNK), jnp.int32),
            pltpu.VMEM((3, CHUNK), jnp.int32),
            pltpu.VMEM((2, CHUNK, D), jnp.float32),
            pltpu.VMEM_SHARED((N_PAD, D), jnp.float32),
            pltpu.SemaphoreType.DMA,
            pltpu.SemaphoreType.DMA,
            pltpu.SemaphoreType.DMA,
        ],
    )(functools.partial(_scatter_body, n0, n1, rem))


def _pre_body(x_ref, w_ref, histt_ref, g_ref):
    h = lax.dot_general(x_ref[...], w_ref[...],
                        (((1,), (1,)), ((), ())),
                        preferred_element_type=jnp.float32)
    deg = histt_ref[:, 0:1] + histt_ref[:, 1:2] + 1.0
    g_ref[...] = h * lax.rsqrt(deg)


_BLK = 1000
_GRID = N_NODES // _BLK


def _tc_pre(x, W, histt):
    return pl.pallas_call(
        _pre_body,
        grid=(_GRID,),
        in_specs=[
            pl.BlockSpec((_BLK, D), lambda i: (i, 0)),
            pl.BlockSpec((D, D), lambda i: (0, 0)),
            pl.BlockSpec((_BLK, 2), lambda i: (i, 0)),
        ],
        out_specs=pl.BlockSpec((_BLK, D), lambda i: (i, 0)),
        out_shape=jax.ShapeDtypeStruct((N_PAD, D), jnp.float32),
    )(x, W, histt)


def _post_body(parts_ref, g_ref, histt_ref, b_ref, out_ref):
    acc = parts_ref[0] + parts_ref[1] + g_ref[...]
    deg = histt_ref[:, 0:1] + histt_ref[:, 1:2] + 1.0
    out = acc * lax.rsqrt(deg) + b_ref[...]
    out_ref[...] = jnp.maximum(out, 0.0)


def _tc_post(parts, g, histt, b2d):
    return pl.pallas_call(
        _post_body,
        grid=(_GRID,),
        in_specs=[
            pl.BlockSpec((2, _BLK, D), lambda i: (0, i, 0)),
            pl.BlockSpec((_BLK, D), lambda i: (i, 0)),
            pl.BlockSpec((_BLK, 2), lambda i: (i, 0)),
            pl.BlockSpec((1, D), lambda i: (0, 0)),
        ],
        out_specs=pl.BlockSpec((_BLK, D), lambda i: (i, 0)),
        out_shape=jax.ShapeDtypeStruct((N_NODES, D), jnp.float32),
    )(parts, g, histt, b2d)


_SLOW_SHARE = 78 / 157


def kernel(x, edge_index, W, b):
    ei = edge_index.astype(jnp.int32)
    E = ei.shape[1]

    if E % CHUNK:
        pad_n = CHUNK - E % CHUNK
        pad_dst = N_NODES + jnp.arange(pad_n, dtype=jnp.int32) % (N_PAD - N_NODES)
        ei = jnp.concatenate(
            [ei, jnp.stack([jnp.zeros((pad_n,), jnp.int32), pad_dst])], axis=1)

    totc = ei.shape[1] // CHUNK
    ei3 = ei.reshape(2, totc, CHUNK)
    per_tile = totc // NS
    rem = totc - NS * per_tile
    n0 = max(1, round(per_tile * _SLOW_SHARE))
    n1 = per_tile - n0

    hist = _make_hist(totc)(ei3)
    histt = hist.T
    g = _tc_pre(x, W, histt)
    parts = _make_scatter(n0, n1, rem)(ei3, g)
    return _tc_post(parts, g, histt, b.reshape(1, D))
